# Initial kernel scaffold; baseline (speedup 1.0000x reference)
import functools

import jax
import jax.numpy as jnp
from jax import lax
from jax.experimental import pallas as pl
from jax.experimental.pallas import tpu as pltpu
from jax.experimental.pallas import tpu_sc as plsc

NC, NS = 2, 16
NW = NC * NS
CSZ = 100


def _make_embed(BL, D, L, n_chunks):
    mesh = plsc.VectorSubcoreMesh(
        core_axis_name="c", subcore_axis_name="s", num_cores=NC, num_subcores=NS
    )
    rows_per_w = BL // NW

    @functools.partial(
        pl.kernel,
        out_type=jax.ShapeDtypeStruct((BL, D), jnp.float32),
        mesh=mesh,
        scratch_types=[
            pltpu.VMEM((n_chunks, CSZ), jnp.int32),
            pltpu.VMEM((L, D), jnp.float32),
            pltpu.VMEM((CSZ, D), jnp.float32),
        ],
    )
    def k(idx_hbm, word_hbm, pos_hbm, out_hbm, idx_v, pos_v, buf):
        wid = lax.axis_index("s") * NC + lax.axis_index("c")
        base = wid * rows_per_w
        pltpu.sync_copy(idx_hbm.at[wid], idx_v)
        pltpu.sync_copy(pos_hbm, pos_v)

        def chunk(c, _):
            off = (c * CSZ) % L
            pltpu.sync_copy(pos_v.at[pl.ds(off, CSZ)], buf)
            pltpu.sync_copy(word_hbm.at[idx_v.at[c]], buf, add=True)
            pltpu.sync_copy(buf, out_hbm.at[pl.ds(base + c * CSZ, CSZ)])
            return 0

        lax.fori_loop(0, n_chunks, chunk, 0)

    return k


def kernel(sequence, sequence_char, word_table, pos_table):
    del sequence_char
    B, L = sequence.shape
    D = word_table.shape[1]
    BL = B * L
    rows_per_w = BL // NW
    n_chunks = rows_per_w // CSZ
    idx = sequence.astype(jnp.int32).reshape(NW, n_chunks, CSZ)
    out = _make_embed(BL, D, L, n_chunks)(
        idx, word_table, pos_table[:L].astype(jnp.float32)
    )
    return out.reshape(B, L, D)


# speedup vs baseline: 2.6203x; 2.6203x over previous
import functools

import jax
import jax.numpy as jnp
from jax import lax
from jax.experimental import pallas as pl
from jax.experimental.pallas import tpu as pltpu
from jax.experimental.pallas import tpu_sc as plsc

NC, NS = 2, 16
NW = NC * NS
CSZ = 100


def _make_embed(BL, D, L, n_chunks):
    mesh = plsc.VectorSubcoreMesh(
        core_axis_name="c", subcore_axis_name="s", num_cores=NC, num_subcores=NS
    )
    rows_per_w = BL // NW

    @functools.partial(
        pl.kernel,
        out_type=jax.ShapeDtypeStruct((BL, D), jnp.float32),
        mesh=mesh,
        scratch_types=[
            pltpu.VMEM((n_chunks, CSZ), jnp.int32),
            pltpu.VMEM((CSZ, D), jnp.float32),
        ],
        compiler_params=pltpu.CompilerParams(use_tc_tiling_on_sc=False),
    )
    def k(idx_hbm, word_hbm, pos_hbm, out_hbm, idx_v, buf):
        wid = lax.axis_index("s") * NC + lax.axis_index("c")
        base = wid * rows_per_w
        pltpu.sync_copy(idx_hbm.at[wid], idx_v)

        def chunk(c, _):
            off = (c * CSZ) % L
            pltpu.sync_copy(pos_hbm.at[pl.ds(off, CSZ)], buf)
            pltpu.sync_copy(word_hbm.at[idx_v.at[c]], buf, add=True)
            pltpu.sync_copy(buf, out_hbm.at[pl.ds(base + c * CSZ, CSZ)])
            return 0

        lax.fori_loop(0, n_chunks, chunk, 0)

    return k


def kernel(sequence, sequence_char, word_table, pos_table):
    del sequence_char
    B, L = sequence.shape
    D = word_table.shape[1]
    BL = B * L
    rows_per_w = BL // NW
    n_chunks = rows_per_w // CSZ
    idx = sequence.astype(jnp.int32).reshape(NW, n_chunks, CSZ)
    out = _make_embed(BL, D, L, n_chunks)(
        idx, word_table, pos_table[:L].astype(jnp.float32)
    )
    return out.reshape(B, L, D)


# speedup vs baseline: 2.6672x; 1.0179x over previous
import functools

import jax
import jax.numpy as jnp
from jax import lax
from jax.experimental import pallas as pl
from jax.experimental.pallas import tpu as pltpu
from jax.experimental.pallas import tpu_sc as plsc

NC, NS = 2, 16
NW = NC * NS
CSZ = 100
GRP = 2
NBANK = 3
NBUF = GRP * NBANK


def _make_embed(BL, D, L, n_chunks):
    mesh = plsc.VectorSubcoreMesh(
        core_axis_name="c", subcore_axis_name="s", num_cores=NC, num_subcores=NS
    )
    rows_per_w = BL // NW
    n_groups = n_chunks // GRP

    @functools.partial(
        pl.kernel,
        out_type=jax.ShapeDtypeStruct((BL, D), jnp.float32),
        mesh=mesh,
        scratch_types=[
            pltpu.VMEM((n_chunks, CSZ), jnp.int32),
            pltpu.VMEM((NBUF, CSZ, D), jnp.float32),
            pltpu.SemaphoreType.DMA((NBUF,)),
            pltpu.SemaphoreType.DMA((NBUF,)),
            pltpu.SemaphoreType.DMA((NBUF,)),
        ],
        compiler_params=pltpu.CompilerParams(use_tc_tiling_on_sc=False),
    )
    def k(idx_hbm, word_hbm, pos_hbm, out_hbm, idx_v, bufs, isem, gsem, osem):
        wid = lax.axis_index("s") * NC + lax.axis_index("c")
        base = wid * rows_per_w
        pltpu.sync_copy(idx_hbm.at[wid], idx_v)

        def issue_i(c, b):
            off = (c * CSZ) % L
            pltpu.async_copy(pos_hbm.at[pl.ds(off, CSZ)], bufs.at[b], isem.at[b])

        def wait_i(b):
            pltpu.make_async_copy(
                pos_hbm.at[pl.ds(0, CSZ)], bufs.at[b], isem.at[b]
            ).wait()

        def issue_g(c, b):
            pltpu.async_copy(
                word_hbm.at[idx_v.at[c]], bufs.at[b], gsem.at[b], add=True
            )

        def wait_g(b):
            pltpu.make_async_copy(
                word_hbm.at[idx_v.at[0]], bufs.at[b], gsem.at[b]
            ).wait()

        def issue_o(c, b):
            pltpu.async_copy(
                bufs.at[b], out_hbm.at[pl.ds(base + c * CSZ, CSZ)], osem.at[b]
            )

        def wait_o(b):
            pltpu.make_async_copy(
                bufs.at[b], out_hbm.at[pl.ds(base, CSZ)], osem.at[b]
            ).wait()

        def step(g, r):
            @pl.when(jnp.logical_and(g >= 2, g <= n_groups + 1))
            def _():
                bank = ((r + 1) % NBANK) * GRP
                for j in range(GRP):
                    wait_g(bank + j)
                    issue_o((g - 2) * GRP + j, bank + j)

            @pl.when(jnp.logical_and(g >= 1, g <= n_groups))
            def _():
                bank = ((r + 2) % NBANK) * GRP
                for j in range(GRP):
                    wait_i(bank + j)
                    issue_g((g - 1) * GRP + j, bank + j)

            @pl.when(jnp.logical_and(g >= NBANK, g < n_groups))
            def _():
                for j in range(GRP):
                    wait_o(r * GRP + j)

            @pl.when(g < n_groups)
            def _():
                for j in range(GRP):
                    issue_i(g * GRP + j, r * GRP + j)

        n_steps = n_groups + 2
        n_iters = -(-n_steps // NBANK)

        def body(h, _):
            for r in range(NBANK):
                step(h * NBANK + r, r)
            return 0

        lax.fori_loop(0, n_iters, body, 0)
        for b in range(NBUF):
            wait_o(b)

    return k


def kernel(sequence, sequence_char, word_table, pos_table):
    del sequence_char
    B, L = sequence.shape
    D = word_table.shape[1]
    BL = B * L
    rows_per_w = BL // NW
    n_chunks = rows_per_w // CSZ
    idx = sequence.astype(jnp.int32).reshape(NW, n_chunks, CSZ)
    out = _make_embed(BL, D, L, n_chunks)(
        idx, word_table, pos_table[:L].astype(jnp.float32)
    )
    return out.reshape(B, L, D)


# speedup vs baseline: 7.5371x; 2.8259x over previous
import functools

import jax
import jax.numpy as jnp
from jax import lax
from jax.experimental import pallas as pl
from jax.experimental.pallas import tpu as pltpu
from jax.experimental.pallas import tpu_sc as plsc

NC, NS = 2, 16
NW = NC * NS
CSZ = 100
NBUF = 6
GLEAD = 3


def _make_embed(BL, D, L, n_chunks):
    mesh = plsc.VectorSubcoreMesh(
        core_axis_name="c", subcore_axis_name="s", num_cores=NC, num_subcores=NS
    )
    rows_per_w = BL // NW
    nvec = D // 16

    @functools.partial(
        pl.kernel,
        out_type=jax.ShapeDtypeStruct((BL, D), jnp.float32),
        mesh=mesh,
        scratch_types=[
            pltpu.VMEM((n_chunks, CSZ), jnp.int32),
            pltpu.VMEM((L, D), jnp.float32),
            pltpu.VMEM((NBUF, CSZ, D), jnp.float32),
            pltpu.SemaphoreType.DMA((NBUF,)),
            pltpu.SemaphoreType.DMA((NBUF,)),
        ],
        compiler_params=pltpu.CompilerParams(use_tc_tiling_on_sc=False),
    )
    def k(idx_hbm, word_hbm, pos_hbm, out_hbm, idx_v, pos_v, bufs, gsem, osem):
        wid = lax.axis_index("s") * NC + lax.axis_index("c")
        base = wid * rows_per_w
        pltpu.sync_copy(idx_hbm.at[wid], idx_v)
        pltpu.sync_copy(pos_hbm, pos_v)

        def issue_g(c, b):
            pltpu.async_copy(word_hbm.at[idx_v.at[c]], bufs.at[b], gsem.at[b])

        def wait_g(b):
            pltpu.make_async_copy(
                word_hbm.at[idx_v.at[0]], bufs.at[b], gsem.at[b]
            ).wait()

        def issue_o(c, b):
            pltpu.async_copy(
                bufs.at[b], out_hbm.at[pl.ds(base + c * CSZ, CSZ)], osem.at[b]
            )

        def wait_o(b):
            pltpu.make_async_copy(
                bufs.at[b], out_hbm.at[pl.ds(base, CSZ)], osem.at[b]
            ).wait()

        def add_pos(c, b):
            off = (c * CSZ) % L

            def row(r, _):
                for j in range(nvec):
                    x = pos_v[off + r, pl.ds(j * 16, 16)]
                    plsc.addupdate(bufs.at[b, r, pl.ds(j * 16, 16)], x)
                return 0

            lax.fori_loop(0, CSZ, row, 0)

        for b in range(GLEAD):
            issue_g(b, b)

        def chunk(c, b):
            wait_g(b)
            add_pos(c, b)
            issue_o(c, b)
            b2 = (b + GLEAD) % NBUF

            @pl.when(jnp.logical_and(c >= GLEAD, c + GLEAD < n_chunks))
            def _():
                wait_o(b2)

            @pl.when(c + GLEAD < n_chunks)
            def _():
                issue_g(c + GLEAD, b2)

        n_iters = -(-n_chunks // NBUF)

        def body(h, _):
            for r in range(NBUF):
                c = h * NBUF + r

                @pl.when(c < n_chunks)
                def _():
                    chunk(c, r)

            return 0

        lax.fori_loop(0, n_iters, body, 0)
        for b in range(NBUF):
            wait_o(b)

    return k


def kernel(sequence, sequence_char, word_table, pos_table):
    del sequence_char
    B, L = sequence.shape
    D = word_table.shape[1]
    BL = B * L
    rows_per_w = BL // NW
    n_chunks = rows_per_w // CSZ
    idx = sequence.astype(jnp.int32).reshape(NW, n_chunks, CSZ)
    out = _make_embed(BL, D, L, n_chunks)(
        idx, word_table, pos_table[:L].astype(jnp.float32)
    )
    return out.reshape(B, L, D)
